# baseline (speedup 1.0000x reference)
import functools

import jax
import jax.numpy as jnp
from jax import lax
from jax.experimental import pallas as pl
from jax.experimental.pallas import tpu as pltpu
from jax.experimental.pallas import tpu_sc as plsc

S = 512
NPIX = S * S
N = 2_000_000
NWORKER = 32
NQUAD = 4
NCHUNK = NWORKER // NQUAD
CHUNK = N // NCHUNK
BLK = 2000
NBLK = CHUNK // BLK
VPB = BLK // 16
QPIX = NPIX // NQUAD
OWN = NPIX // NWORKER
HALF = OWN // 2
GCH = 128

_mesh = plsc.VectorSubcoreMesh(core_axis_name="c", subcore_axis_name="s")
_params = pltpu.CompilerParams(needs_layout_passes=False)


def _stage1(pos, part, posbuf, winner, nbscr):
    wid = lax.axis_index("s") * 2 + lax.axis_index("c")
    chunk = wid // NQUAD
    q = wid % NQUAD
    iota = lax.iota(jnp.int32, 16)
    iota3 = iota * 3
    nbscr[pl.ds(16, 16)] = jnp.full((16,), -16, jnp.int32)

    def initw(j, _):
        winner[pl.ds(j * 16, 16)] = jnp.full((16,), -1, jnp.int32)
        return 0

    lax.fori_loop(0, QPIX // 16, initw, 0)

    def blk_body(b, _):
        start = (chunk * CHUNK + b * BLK) * 3
        pltpu.sync_copy(pos.at[pl.ds(start, BLK * 3)], posbuf)

        def v_body(i, _):
            r3 = i * 48 + iota3
            x = plsc.load_gather(posbuf, [r3])
            y = plsc.load_gather(posbuf, [r3 + 1])
            ix = ((x + 1.0) * (S / 2.0)).astype(jnp.int32)
            iy = ((y + 1.0) * (S / 2.0)).astype(jnp.int32)
            ix = jnp.clip(ix, 0, S - 1)
            iy = jnp.clip(iy, 0, S - 1)
            flat = iy * S + ix
            key = flat * 16 + iota
            sk = plsc.sort_key_val(key, key)[0]
            nbscr[pl.ds(0, 16)] = sk
            nb = nbscr[pl.ds(1, 16)]
            skf = sk >> 4
            mask = jnp.logical_and((nb >> 4) != skf, (skf >> 16) == q)
            val = (b * BLK + i * 16) + (sk & 15)
            plsc.store_scatter(winner, [skf & (QPIX - 1)], val, mask=mask)
            return 0

        lax.fori_loop(0, VPB, v_body, 0)
        return 0

    lax.fori_loop(0, NBLK, blk_body, 0)
    pltpu.sync_copy(winner, part.at[wid])


_stage1_call = pl.kernel(
    _stage1,
    out_type=jax.ShapeDtypeStruct((NWORKER, QPIX), jnp.int32),
    mesh=_mesh,
    compiler_params=_params,
    scratch_types=[
        pltpu.VMEM((BLK * 3,), jnp.float32),
        pltpu.VMEM((QPIX,), jnp.int32),
        pltpu.VMEM((32,), jnp.int32),
    ],
)


def _stage2(part, colors, opac, out, pbuf, merged, cidx, cplane, obuf, outbuf, sem):
    wid = lax.axis_index("s") * 2 + lax.axis_index("c")
    q = wid // 8
    sub = wid % 8

    for h in range(2):
        pixbase = wid * OWN + h * HALF
        col0 = sub * OWN + h * HALF
        for c in range(NCHUNK):
            pltpu.sync_copy(part.at[c * NQUAD + q, pl.ds(col0, HALF)],
                            pbuf.at[pl.ds(c * HALF, HALF)])

        def mbody(j, _):
            m = jnp.full((16,), -1, jnp.int32)
            for c in range(NCHUNK):
                lc = pbuf[pl.ds(c * HALF + j * 16, 16)]
                g = jnp.where(lc >= 0, c * CHUNK + lc, -1)
                m = jnp.maximum(m, g)
            merged[pl.ds(j * 16, 16)] = m
            w = jnp.maximum(m, 0)
            w3 = w * 3
            cidx[pl.ds(j * 16, 16)] = w3
            cidx[pl.ds(HALF + j * 16, 16)] = w3 + 1
            cidx[pl.ds(2 * HALF + j * 16, 16)] = w3 + 2
            cidx[pl.ds(3 * HALF + j * 16, 16)] = w
            return 0

        lax.fori_loop(0, HALF // 16, mbody, 0)

        cps = []
        for ch in range(3):
            for j2 in range(HALF // GCH):
                o = ch * HALF + j2 * GCH
                cps.append(pltpu.async_copy(colors.at[cidx.at[pl.ds(o, GCH)]],
                                            cplane.at[pl.ds(o, GCH)], sem))
        for j2 in range(HALF // GCH):
            o = 3 * HALF + j2 * GCH
            cps.append(pltpu.async_copy(opac.at[cidx.at[pl.ds(o, GCH)]],
                                        obuf.at[pl.ds(j2 * GCH, GCH)], sem))
        for cp in cps:
            cp.wait()

        def obody(j, _):
            o = obuf[pl.ds(j * 16, 16)]
            valid = merged[pl.ds(j * 16, 16)] >= 0
            oz = jnp.where(valid, o, 0.0)
            for ch in range(3):
                cv = cplane[pl.ds(ch * HALF + j * 16, 16)]
                outbuf[pl.ds(ch * HALF + j * 16, 16)] = cv * oz
            outbuf[pl.ds(3 * HALF + j * 16, 16)] = oz
            return 0

        lax.fori_loop(0, HALF // 16, obody, 0)

        for ch in range(4):
            pltpu.sync_copy(outbuf.at[pl.ds(ch * HALF, HALF)],
                            out.at[ch, pl.ds(pixbase, HALF)])


_stage2_call = pl.kernel(
    _stage2,
    out_type=jax.ShapeDtypeStruct((4, NPIX), jnp.float32),
    mesh=_mesh,
    compiler_params=_params,
    scratch_types=[
        pltpu.VMEM((NCHUNK * HALF,), jnp.int32),
        pltpu.VMEM((HALF,), jnp.int32),
        pltpu.VMEM((4 * HALF,), jnp.int32),
        pltpu.VMEM((3 * HALF,), jnp.float32),
        pltpu.VMEM((HALF,), jnp.float32),
        pltpu.VMEM((4 * HALF,), jnp.float32),
        pltpu.SemaphoreType.DMA,
    ],
)


@jax.jit
def kernel(positions, colors, opacities, camera_params):
    part = _stage1_call(positions.reshape(-1))
    img = _stage2_call(part, colors.reshape(-1), opacities)
    return img.reshape(1, 4, S, S)


# speedup vs baseline: 15.9425x; 15.9425x over previous
import functools

import jax
import jax.numpy as jnp
from jax import lax
from jax.experimental import pallas as pl
from jax.experimental.pallas import tpu as pltpu
from jax.experimental.pallas import tpu_sc as plsc

S = 512
NPIX = S * S
N = 2_000_000
NWORKER = 32
NQUAD = 4
NCHUNK = NWORKER // NQUAD
CHUNK = N // NCHUNK
BLK = 2000
NBLK = CHUNK // BLK
VPB = BLK // 16
UNROLL = 5
QPIX = NPIX // NQUAD
OWN = NPIX // NWORKER
HALF = OWN // 2
GCH = 128
ROWS = N // 128
BR = 125

_mesh = plsc.VectorSubcoreMesh(core_axis_name="c", subcore_axis_name="s")
_params = pltpu.CompilerParams(needs_layout_passes=False)


def _prep(x_ref, y_ref, r_ref, g_ref, b_ref, o_ref,
          flat_ref, rp_ref, gp_ref, bp_ref, ap_ref):
    ix = ((x_ref[...] + 1.0) * (S / 2.0)).astype(jnp.int32)
    iy = ((y_ref[...] + 1.0) * (S / 2.0)).astype(jnp.int32)
    ix = jnp.clip(ix, 0, S - 1)
    iy = jnp.clip(iy, 0, S - 1)
    flat_ref[...] = iy * S + ix
    o = o_ref[...]
    rp_ref[...] = r_ref[...] * o
    gp_ref[...] = g_ref[...] * o
    bp_ref[...] = b_ref[...] * o
    ap_ref[...] = o


PBLK = 16384
_block = pl.BlockSpec((PBLK,), lambda i: (i,))
_prep_call = pl.pallas_call(
    _prep,
    grid=(pl.cdiv(N, PBLK),),
    in_specs=[_block] * 6,
    out_specs=[_block] * 5,
    out_shape=[jax.ShapeDtypeStruct((N,), jnp.int32)]
    + [jax.ShapeDtypeStruct((N,), jnp.float32)] * 4,
)


def _vshift_up(v, nidx):
    dnums = lax.GatherDimensionNumbers(
        offset_dims=(), collapsed_slice_dims=(0,), start_index_map=(0,))
    return lax.gather(v, nidx[:, None], dnums, (1,),
                      mode=lax.GatherScatterMode.PROMISE_IN_BOUNDS)


def _stage1(flat, part, kbuf, winner):
    wid = lax.axis_index("s") * 2 + lax.axis_index("c")
    chunk = wid // NQUAD
    q = wid % NQUAD
    iota = lax.iota(jnp.int32, 16)
    nidx = jnp.minimum(iota + 1, 15)
    l15 = iota == 15

    def initw(j, _):
        winner[pl.ds(j * 16, 16)] = jnp.full((16,), -1, jnp.int32)
        return 0

    lax.fori_loop(0, QPIX // 16, initw, 0)

    def blk_body(b, _):
        pltpu.sync_copy(flat.at[pl.ds(chunk * CHUNK + b * BLK, BLK)], kbuf)

        def v_body(i, _):
            for u in range(UNROLL):
                iu = i * UNROLL + u
                k = kbuf[pl.ds(iu * 16, 16)]
                key = k * 16 + iota
                sk = plsc.sort_key_val(key, iota)[0]
                nb = _vshift_up(sk, nidx)
                skf = sk >> 4
                mask = jnp.logical_and(
                    jnp.logical_or((nb >> 4) != skf, l15),
                    (skf >> 16) == q)
                val = (b * BLK + iu * 16) + (sk & 15)
                plsc.store_scatter(winner, [skf & (QPIX - 1)], val, mask=mask)
            return 0

        lax.fori_loop(0, VPB // UNROLL, v_body, 0)
        return 0

    lax.fori_loop(0, NBLK, blk_body, 0)
    pltpu.sync_copy(winner, part.at[wid])


_stage1_call = pl.kernel(
    _stage1,
    out_type=jax.ShapeDtypeStruct((NWORKER, QPIX), jnp.int32),
    mesh=_mesh,
    compiler_params=_params,
    scratch_types=[
        pltpu.VMEM((BLK,), jnp.int32),
        pltpu.VMEM((QPIX,), jnp.int32),
    ],
)


def _stage2(part, rp, gp, bp, ap, out, pbuf, merged, widx, gbuf, outbuf, sem):
    wid = lax.axis_index("s") * 2 + lax.axis_index("c")
    q = wid // 8
    sub = wid % 8

    for h in range(2):
        pixbase = wid * OWN + h * HALF
        col0 = sub * OWN + h * HALF
        for c in range(NCHUNK):
            pltpu.sync_copy(part.at[c * NQUAD + q, pl.ds(col0, HALF)],
                            pbuf.at[pl.ds(c * HALF, HALF)])

        def mbody(j, _):
            m = jnp.full((16,), -1, jnp.int32)
            for c in range(NCHUNK):
                lc = pbuf[pl.ds(c * HALF + j * 16, 16)]
                g = jnp.where(lc >= 0, c * CHUNK + lc, -1)
                m = jnp.maximum(m, g)
            merged[pl.ds(j * 16, 16)] = m
            widx[pl.ds(j * 16, 16)] = jnp.maximum(m, 0)
            return 0

        lax.fori_loop(0, HALF // 16, mbody, 0)

        cps = []
        for pi, src in enumerate((rp, gp, bp, ap)):
            for j2 in range(HALF // GCH):
                cps.append(pltpu.async_copy(
                    src.at[widx.at[pl.ds(j2 * GCH, GCH)]],
                    gbuf.at[pl.ds(pi * HALF + j2 * GCH, GCH)], sem))
        for cp in cps:
            cp.wait()

        def obody(j, _):
            valid = merged[pl.ds(j * 16, 16)] >= 0
            for ch in range(4):
                v = gbuf[pl.ds(ch * HALF + j * 16, 16)]
                outbuf[pl.ds(ch * HALF + j * 16, 16)] = jnp.where(valid, v, 0.0)
            return 0

        lax.fori_loop(0, HALF // 16, obody, 0)

        for ch in range(4):
            pltpu.sync_copy(outbuf.at[pl.ds(ch * HALF, HALF)],
                            out.at[ch, pl.ds(pixbase, HALF)])


_stage2_call = pl.kernel(
    _stage2,
    out_type=jax.ShapeDtypeStruct((4, NPIX), jnp.float32),
    mesh=_mesh,
    compiler_params=_params,
    scratch_types=[
        pltpu.VMEM((NCHUNK * HALF,), jnp.int32),
        pltpu.VMEM((HALF,), jnp.int32),
        pltpu.VMEM((HALF,), jnp.int32),
        pltpu.VMEM((4 * HALF,), jnp.float32),
        pltpu.VMEM((4 * HALF,), jnp.float32),
        pltpu.SemaphoreType.DMA,
    ],
)


@jax.jit
def kernel(positions, colors, opacities, camera_params):
    flat, rp, gp, bp, ap = _prep_call(
        positions[:, 0], positions[:, 1],
        colors[:, 0], colors[:, 1], colors[:, 2], opacities)
    part = _stage1_call(flat)
    img = _stage2_call(part, rp, gp, bp, ap)
    return img.reshape(1, 4, S, S)


# speedup vs baseline: 26.3677x; 1.6539x over previous
import functools

import jax
import jax.numpy as jnp
from jax import lax
from jax.experimental import pallas as pl
from jax.experimental.pallas import tpu as pltpu
from jax.experimental.pallas import tpu_sc as plsc

S = 512
NPIX = S * S
N = 2_000_000
NWORKER = 32
NQUAD = 4
NCHUNK = NWORKER // NQUAD
CHUNK = N // NCHUNK
BLK = 10000
NBLK = CHUNK // BLK
VPB = BLK // 16
UNROLL = 5
QPIX = NPIX // NQUAD
OWN = NPIX // NWORKER
HALF = OWN // 2
GCH = 128
ROWS = N // 128
BR = 125

_mesh = plsc.VectorSubcoreMesh(core_axis_name="c", subcore_axis_name="s")
_params = pltpu.CompilerParams(needs_layout_passes=False)


def _prep(x_ref, y_ref, r_ref, g_ref, b_ref, o_ref,
          flat_ref, rp_ref, gp_ref, bp_ref, ap_ref):
    ix = ((x_ref[...] + 1.0) * (S / 2.0)).astype(jnp.int32)
    iy = ((y_ref[...] + 1.0) * (S / 2.0)).astype(jnp.int32)
    ix = jnp.clip(ix, 0, S - 1)
    iy = jnp.clip(iy, 0, S - 1)
    flat_ref[...] = iy * S + ix
    o = o_ref[...]
    rp_ref[...] = r_ref[...] * o
    gp_ref[...] = g_ref[...] * o
    bp_ref[...] = b_ref[...] * o
    ap_ref[...] = o


PBLK = 16384
_block = pl.BlockSpec((PBLK,), lambda i: (i,))
_prep_call = pl.pallas_call(
    _prep,
    grid=(pl.cdiv(N, PBLK),),
    in_specs=[_block] * 6,
    out_specs=[_block] * 5,
    out_shape=[jax.ShapeDtypeStruct((N,), jnp.int32)]
    + [jax.ShapeDtypeStruct((N,), jnp.float32)] * 4,
)


def _vshift_up(v, nidx):
    dnums = lax.GatherDimensionNumbers(
        offset_dims=(), collapsed_slice_dims=(0,), start_index_map=(0,))
    return lax.gather(v, nidx[:, None], dnums, (1,),
                      mode=lax.GatherScatterMode.PROMISE_IN_BOUNDS)


def _stage1(flat, part, kbuf, winner, sem):
    wid = lax.axis_index("s") * 2 + lax.axis_index("c")
    chunk = wid // NQUAD
    q = wid % NQUAD
    iota = lax.iota(jnp.int32, 16)
    nidx = jnp.minimum(iota + 1, 15)
    l15 = iota == 15

    def initw(j, _):
        winner[pl.ds(j * 16, 16)] = jnp.full((16,), -1, jnp.int32)
        return 0

    lax.fori_loop(0, QPIX // 16, initw, 0)

    def fetch(b):
        return pltpu.async_copy(
            flat.at[pl.ds(chunk * CHUNK + b * BLK, BLK)],
            kbuf.at[pl.ds((b % 2) * BLK, BLK)], sem)

    pending = fetch(0)
    for b in range(NBLK):
        pending.wait()
        if b + 1 < NBLK:
            pending = fetch(b + 1)
        boff = (b % 2) * BLK

        def v_body(i, _, b=b, boff=boff):
            base = boff + i * (16 * UNROLL)
            ks = [kbuf[pl.ds(base + u * 16, 16)] for u in range(UNROLL)]
            sks = [
                plsc.bitcast(
                    plsc.sort_key_val(
                        plsc.bitcast(k * 16 + iota, jnp.uint32), iota)[0],
                    jnp.int32)
                for k in ks
            ]
            nbs = [_vshift_up(sk, nidx) for sk in sks]
            for u in range(UNROLL):
                sk = sks[u]
                skf = sk >> 4
                mask = jnp.logical_and(
                    jnp.logical_or((nbs[u] >> 4) != skf, l15),
                    (skf >> 16) == q)
                val = (b * BLK + i * (16 * UNROLL) + u * 16) + (sk & 15)
                plsc.store_scatter(winner, [skf & (QPIX - 1)], val, mask=mask)
            return 0

        lax.fori_loop(0, VPB // UNROLL, v_body, 0)

    pltpu.sync_copy(winner, part.at[wid])


_stage1_call = pl.kernel(
    _stage1,
    out_type=jax.ShapeDtypeStruct((NWORKER, QPIX), jnp.int32),
    mesh=_mesh,
    compiler_params=_params,
    scratch_types=[
        pltpu.VMEM((2 * BLK,), jnp.int32),
        pltpu.VMEM((QPIX,), jnp.int32),
        pltpu.SemaphoreType.DMA,
    ],
)


def _stage2(part, rp, gp, bp, ap, out, pbuf, merged, widx, gbuf, outbuf, sem):
    wid = lax.axis_index("s") * 2 + lax.axis_index("c")
    q = wid // 8
    sub = wid % 8

    for h in range(2):
        pixbase = wid * OWN + h * HALF
        col0 = sub * OWN + h * HALF
        for c in range(NCHUNK):
            pltpu.sync_copy(part.at[c * NQUAD + q, pl.ds(col0, HALF)],
                            pbuf.at[pl.ds(c * HALF, HALF)])

        def mbody(j, _):
            m = jnp.full((16,), -1, jnp.int32)
            for c in range(NCHUNK):
                lc = pbuf[pl.ds(c * HALF + j * 16, 16)]
                g = jnp.where(lc >= 0, c * CHUNK + lc, -1)
                m = jnp.maximum(m, g)
            merged[pl.ds(j * 16, 16)] = m
            widx[pl.ds(j * 16, 16)] = jnp.maximum(m, 0)
            return 0

        lax.fori_loop(0, HALF // 16, mbody, 0)

        cps = []
        for pi, src in enumerate((rp, gp, bp, ap)):
            for j2 in range(HALF // GCH):
                cps.append(pltpu.async_copy(
                    src.at[widx.at[pl.ds(j2 * GCH, GCH)]],
                    gbuf.at[pl.ds(pi * HALF + j2 * GCH, GCH)], sem))
        for cp in cps:
            cp.wait()

        def obody(j, _):
            valid = merged[pl.ds(j * 16, 16)] >= 0
            for ch in range(4):
                v = gbuf[pl.ds(ch * HALF + j * 16, 16)]
                outbuf[pl.ds(ch * HALF + j * 16, 16)] = jnp.where(valid, v, 0.0)
            return 0

        lax.fori_loop(0, HALF // 16, obody, 0)

        for ch in range(4):
            pltpu.sync_copy(outbuf.at[pl.ds(ch * HALF, HALF)],
                            out.at[ch, pl.ds(pixbase, HALF)])


_stage2_call = pl.kernel(
    _stage2,
    out_type=jax.ShapeDtypeStruct((4, NPIX), jnp.float32),
    mesh=_mesh,
    compiler_params=_params,
    scratch_types=[
        pltpu.VMEM((NCHUNK * HALF,), jnp.int32),
        pltpu.VMEM((HALF,), jnp.int32),
        pltpu.VMEM((HALF,), jnp.int32),
        pltpu.VMEM((4 * HALF,), jnp.float32),
        pltpu.VMEM((4 * HALF,), jnp.float32),
        pltpu.SemaphoreType.DMA,
    ],
)


@jax.jit
def kernel(positions, colors, opacities, camera_params):
    flat, rp, gp, bp, ap = _prep_call(
        positions[:, 0], positions[:, 1],
        colors[:, 0], colors[:, 1], colors[:, 2], opacities)
    part = _stage1_call(flat)
    img = _stage2_call(part, rp, gp, bp, ap)
    return img.reshape(1, 4, S, S)
